# baseline (speedup 1.0000x reference)
import functools

import jax
import jax.numpy as jnp
from jax import lax
from jax.experimental import pallas as pl
from jax.experimental.pallas import tpu as pltpu
from jax.experimental.pallas import tpu_sc as plsc

NW = 32
LANE = 128
K = 4
NBUF = 2


@functools.lru_cache(maxsize=None)
def _build(nb, nf, D):
    B = nb * nf
    nsub = D // 8
    tile = 8 * LANE
    assert D % 16 == 0 and nb % LANE == 0 and B % (NW * K * LANE) == 0
    nblk = B // (NW * LANE)
    ng = nblk // K
    tpj = nb // LANE
    assert ng >= 2 and ng % 2 == 0

    mesh = plsc.VectorSubcoreMesh(core_axis_name="c", subcore_axis_name="s")

    @functools.partial(
        pl.kernel,
        out_type=jax.ShapeDtypeStruct((nf * nsub, tpj * tile), jnp.float32),
        mesh=mesh,
        scratch_types=[
            pltpu.VMEM((nblk * LANE,), jnp.int32),
            pltpu.VMEM((NBUF, K * LANE, D), jnp.float32),
            pltpu.VMEM((NBUF, K, nsub, tile), jnp.float32),
            pltpu.SemaphoreType.DMA((NBUF,)),
            pltpu.SemaphoreType.DMA((NBUF,)),
        ],
        compiler_params=pltpu.CompilerParams(use_tc_tiling_on_sc=False,
                                             needs_layout_passes=False),
    )
    def gather_kernel(table, idx_hbm, out_hbm, idx_v, rows, tiles, gsem, ssem):
        w = lax.axis_index("s") * 2 + lax.axis_index("c")
        pltpu.sync_copy(idx_hbm.at[pl.ds(w * nblk * LANE, nblk * LANE)],
                        idx_v)
        gbase = w * ng
        lanes = lax.broadcasted_iota(jnp.int32, (16,), 0)
        bidxs = [lanes + 16 * k for k in range(K * LANE // 16)]
        fvs = [jnp.full((16,), f, jnp.int32) for f in range(D)]

        def start_gather(g_local, b):
            pltpu.async_copy(
                table.at[idx_v.at[pl.ds(g_local * K * LANE, K * LANE)]],
                rows.at[b], gsem.at[b])

        def wait_gather(b):
            pltpu.make_async_copy(table.at[idx_v.at[pl.ds(0, K * LANE)]],
                                  rows.at[b], gsem.at[b]).wait()

        def transpose(b, u):
            rr = rows.at[b]
            tt = tiles.at[b, u]
            for f0 in range(0, D, 2):
                vs = [plsc.load_gather(
                          rr, [bidxs[u * 8 + (k % 8)], fvs[f0 + (k // 8)]])
                      for k in range(16)]
                for k in range(16):
                    f = f0 + (k // 8)
                    tt[f // 8, pl.ds((f % 8) * LANE + 16 * (k % 8), 16)] = vs[k]

        def process(g, b):
            for u in range(K):
                c = g * K + u
                j = c // tpj
                t = lax.rem(c, tpj)
                transpose(b, u)
                pltpu.async_copy(
                    tiles.at[b, u],
                    out_hbm.at[pl.ds(j * nsub, nsub), pl.ds(t * tile, tile)],
                    ssem.at[b])

        def wait_scatter(b):
            for u in range(K):
                pltpu.make_async_copy(
                    tiles.at[b, u],
                    out_hbm.at[pl.ds(0, nsub), pl.ds(0, tile)],
                    ssem.at[b]).wait()

        start_gather(0, 0)
        wait_gather(0)
        process(gbase, 0)
        start_gather(1, 1)

        @pl.loop(1, ng - 1)
        def _(g_local):
            b = lax.rem(g_local, NBUF)

            @pl.when(g_local >= NBUF)
            def _():
                wait_scatter(b)

            wait_gather(b)
            process(gbase + g_local, b)
            start_gather(g_local + 1, 1 - b)

        b_last = (ng - 1) % NBUF
        wait_scatter(b_last)
        wait_gather(b_last)
        process(gbase + ng - 1, b_last)
        wait_scatter(1 - b_last)
        wait_scatter(b_last)

    return gather_kernel


def kernel(image, gather_indices):
    nb, nf, _ = gather_indices.shape
    B = nb * nf
    D = image.shape[1]
    idx = jnp.transpose(gather_indices, (1, 2, 0)).reshape(B).astype(jnp.int32)
    outb = _build(nb, nf, D)(image, idx)
    out5 = outb.reshape(nf, D // 8, nb // LANE, 8, LANE)
    return jnp.transpose(out5, (2, 4, 0, 1, 3)).reshape(nb, nf, D)


# speedup vs baseline: 1.0432x; 1.0432x over previous
import functools

import jax
import jax.numpy as jnp
from jax import lax
from jax.experimental import pallas as pl
from jax.experimental.pallas import tpu as pltpu
from jax.experimental.pallas import tpu_sc as plsc

NW = 32
LANE = 128
K = 4
NBUF = 2


@functools.lru_cache(maxsize=None)
def _build(nb, nf, D):
    B = nb * nf
    nsub = D // 8
    tile = 8 * LANE
    assert D % 16 == 0 and nb % LANE == 0 and B % (NW * K * LANE) == 0
    nblk = B // (NW * LANE)
    ng = nblk // K
    tpj = nb // LANE
    assert ng >= 2 and ng % 2 == 0

    mesh = plsc.VectorSubcoreMesh(core_axis_name="c", subcore_axis_name="s")

    @functools.partial(
        pl.kernel,
        out_type=jax.ShapeDtypeStruct((nf * nsub, tpj * tile), jnp.float32),
        mesh=mesh,
        scratch_types=[
            pltpu.VMEM((nblk * LANE,), jnp.int32),
            pltpu.VMEM((NBUF, K * LANE, D), jnp.float32),
            pltpu.VMEM((NBUF, K, nsub, tile), jnp.float32),
            pltpu.SemaphoreType.DMA((NBUF,)),
            pltpu.SemaphoreType.DMA((NBUF,)),
        ],
        compiler_params=pltpu.CompilerParams(use_tc_tiling_on_sc=False,
                                             needs_layout_passes=False),
    )
    def gather_kernel(table, idx_hbm, out_hbm, idx_v, rows, tiles, gsem, ssem):
        w = lax.axis_index("s") * 2 + lax.axis_index("c")
        pltpu.sync_copy(idx_hbm.at[pl.ds(w * nblk * LANE, nblk * LANE)],
                        idx_v)
        gbase = w * ng
        lanes = lax.broadcasted_iota(jnp.int32, (16,), 0)
        bidxs = [lanes + 16 * k for k in range(K * LANE // 16)]
        fvs = [jnp.full((16,), f, jnp.int32) for f in range(D)]

        def start_gather(g_local, b):
            pltpu.async_copy(
                table.at[idx_v.at[pl.ds(g_local * K * LANE, K * LANE)]],
                rows.at[b], gsem.at[b])

        def wait_gather(b):
            pltpu.make_async_copy(table.at[idx_v.at[pl.ds(0, K * LANE)]],
                                  rows.at[b], gsem.at[b]).wait()

        def transpose(b, u):
            rr = rows.at[b]
            tt = tiles.at[b, u]
            for f0 in range(0, D, 2):
                vs = [plsc.load_gather(
                          rr, [bidxs[u * 8 + (k % 8)], fvs[f0 + (k // 8)]])
                      for k in range(16)]
                for k in range(16):
                    f = f0 + (k // 8)
                    tt[f // 8, pl.ds((f % 8) * LANE + 16 * (k % 8), 16)] = vs[k]

        def process(g, b):
            for u in range(K):
                c = g * K + u
                j = c // tpj
                t = lax.rem(c, tpj)
                transpose(b, u)
                pltpu.async_copy(
                    tiles.at[b, u],
                    out_hbm.at[pl.ds(j * nsub, nsub), pl.ds(t * tile, tile)],
                    ssem.at[b])

        def wait_scatter(b):
            for u in range(K):
                pltpu.make_async_copy(
                    tiles.at[b, u],
                    out_hbm.at[pl.ds(0, nsub), pl.ds(0, tile)],
                    ssem.at[b]).wait()

        start_gather(0, 0)
        wait_gather(0)
        start_gather(1, 1)
        process(gbase, 0)

        @pl.loop(1, ng - 1)
        def _(g_local):
            b = lax.rem(g_local, NBUF)

            @pl.when(g_local >= NBUF)
            def _():
                wait_scatter(b)

            wait_gather(b)
            start_gather(g_local + 1, 1 - b)
            process(gbase + g_local, b)

        b_last = (ng - 1) % NBUF
        wait_scatter(b_last)
        wait_gather(b_last)
        process(gbase + ng - 1, b_last)
        wait_scatter(1 - b_last)
        wait_scatter(b_last)

    return gather_kernel


def kernel(image, gather_indices):
    nb, nf, _ = gather_indices.shape
    B = nb * nf
    D = image.shape[1]
    idx = jnp.transpose(gather_indices, (1, 2, 0)).reshape(B).astype(jnp.int32)
    outb = _build(nb, nf, D)(image, idx)
    out5 = outb.reshape(nf, D // 8, nb // LANE, 8, LANE)
    return jnp.transpose(out5, (2, 4, 0, 1, 3)).reshape(nb, nf, D)


# speedup vs baseline: 1.4293x; 1.3702x over previous
import functools

import jax
import jax.numpy as jnp
from jax import lax
from jax.experimental import pallas as pl
from jax.experimental.pallas import tpu as pltpu
from jax.experimental.pallas import tpu_sc as plsc

NW = 32
LANE = 128
K = 4
NBUF = 2


@functools.lru_cache(maxsize=None)
def _build(nb, nf, D):
    B = nb * nf
    nsub = D // 8
    tile = 8 * LANE
    assert D % 16 == 0 and nb % LANE == 0 and B % (NW * K * LANE) == 0
    nblk = B // (NW * LANE)
    ng = nblk // K
    tpj = nb // LANE
    assert ng >= 2 and ng % 2 == 0

    mesh = plsc.VectorSubcoreMesh(core_axis_name="c", subcore_axis_name="s")

    @functools.partial(
        pl.kernel,
        out_type=jax.ShapeDtypeStruct((nf * nsub, tpj * tile), jnp.float32),
        mesh=mesh,
        scratch_types=[
            pltpu.VMEM((nblk * LANE,), jnp.int32),
            pltpu.VMEM((NBUF, K * LANE, D), jnp.float32),
            pltpu.VMEM((NBUF, K, nsub, tile), jnp.float32),
            pltpu.SemaphoreType.DMA((NBUF,)),
            pltpu.SemaphoreType.DMA((NBUF,)),
        ],
        compiler_params=pltpu.CompilerParams(use_tc_tiling_on_sc=False,
                                             needs_layout_passes=False),
    )
    def gather_kernel(table, idx_hbm, out_hbm, idx_v, rows, tiles, gsem, ssem):
        w = lax.axis_index("s") * 2 + lax.axis_index("c")
        pltpu.sync_copy(idx_hbm.at[pl.ds(w * nblk * LANE, nblk * LANE)],
                        idx_v)
        gbase = w * ng
        lanes = lax.broadcasted_iota(jnp.int32, (16,), 0)

        def start_gather(g_local, b):
            pltpu.async_copy(
                table.at[idx_v.at[pl.ds(g_local * K * LANE, K * LANE)]],
                rows.at[b], gsem.at[b])

        def wait_gather(b):
            pltpu.make_async_copy(table.at[idx_v.at[pl.ds(0, K * LANE)]],
                                  rows.at[b], gsem.at[b]).wait()

        def transpose(b, u):
            rr = rows.at[b]
            tt = tiles.at[b, u]
            fd = [lax.rem(lanes + d, 16) for d in range(16)]
            wd = [lax.rem(fd[d], 8) * LANE + lanes for d in range(16)]

            @pl.loop(0, LANE // 16)
            def _(k):
                bv = lanes + u * LANE + k * 16
                for f0 in range(0, D, 16):
                    vs = [plsc.load_gather(rr, [bv, fd[d] + f0])
                          for d in range(16)]
                    for d in range(16):
                        plsc.store_scatter(
                            tt, [fd[d] // 8 + f0 // 8, wd[d] + k * 16], vs[d])

        def process(g, b):
            for u in range(K):
                c = g * K + u
                j = c // tpj
                t = lax.rem(c, tpj)
                transpose(b, u)
                pltpu.async_copy(
                    tiles.at[b, u],
                    out_hbm.at[pl.ds(j * nsub, nsub), pl.ds(t * tile, tile)],
                    ssem.at[b])

        def wait_scatter(b):
            for u in range(K):
                pltpu.make_async_copy(
                    tiles.at[b, u],
                    out_hbm.at[pl.ds(0, nsub), pl.ds(0, tile)],
                    ssem.at[b]).wait()

        start_gather(0, 0)

        @pl.loop(0, ng)
        def _(g_local):
            b = lax.rem(g_local, NBUF)

            @pl.when(g_local >= NBUF)
            def _():
                wait_scatter(b)

            wait_gather(b)

            @pl.when(g_local + 1 < ng)
            def _():
                start_gather(g_local + 1, 1 - b)

            process(gbase + g_local, b)

        wait_scatter(lax.rem(ng - 1, NBUF))
        wait_scatter(lax.rem(ng, NBUF))

    return gather_kernel


def kernel(image, gather_indices):
    nb, nf, _ = gather_indices.shape
    B = nb * nf
    D = image.shape[1]
    idx = jnp.transpose(gather_indices, (1, 2, 0)).reshape(B).astype(jnp.int32)
    outb = _build(nb, nf, D)(image, idx)
    out5 = outb.reshape(nf, D // 8, nb // LANE, 8, LANE)
    return jnp.transpose(out5, (2, 4, 0, 1, 3)).reshape(nb, nf, D)


# speedup vs baseline: 2.6215x; 1.8341x over previous
import functools

import jax
import jax.numpy as jnp
from jax import lax
from jax.experimental import pallas as pl
from jax.experimental.pallas import tpu as pltpu
from jax.experimental.pallas import tpu_sc as plsc

NW = 32
LANE = 128
K = 4
NBUF = 2


@functools.lru_cache(maxsize=None)
def _build_repack(F, N):
    NB = N // LANE
    TAIL = N - NB * LANE
    mesh = plsc.VectorSubcoreMesh(core_axis_name="c", subcore_axis_name="s")

    @functools.partial(
        pl.kernel,
        out_type=jax.ShapeDtypeStruct((N * F // LANE, LANE), jnp.float32),
        mesh=mesh,
        scratch_types=[
            pltpu.VMEM((NBUF, F, LANE), jnp.float32),
            pltpu.VMEM((NBUF, F, LANE), jnp.float32),
            pltpu.SemaphoreType.DMA((NBUF,)),
            pltpu.SemaphoreType.DMA((NBUF,)),
        ],
        compiler_params=pltpu.CompilerParams(use_tc_tiling_on_sc=True,
                                             needs_layout_passes=False),
    )
    def repack_kernel(src, tail, dst, vin, vout, gsem, ssem):
        w = lax.axis_index("s") * 2 + lax.axis_index("c")
        nw = NB // NW + jnp.where(w < NB % NW, 1, 0)
        lanes = lax.broadcasted_iota(jnp.int32, (16,), 0)
        fd = [lax.rem(lanes + d, 16) for d in range(16)]
        sq = lanes // 4
        wq = lax.rem(lanes, 4) * F

        def blk(m):
            return m * NW + w

        def start_in(m, b):
            pltpu.async_copy(src.at[:, pl.ds(blk(m) * LANE, LANE)],
                             vin.at[b], gsem.at[b])

        def wait_in(b):
            pltpu.make_async_copy(src.at[:, pl.ds(0, LANE)], vin.at[b],
                                  gsem.at[b]).wait()

        def start_out(m, b):
            pltpu.async_copy(vout.at[b], dst.at[pl.ds(blk(m) * F, F), :],
                             ssem.at[b])

        def wait_out(b):
            pltpu.make_async_copy(vout.at[b], dst.at[pl.ds(0, F), :],
                                  ssem.at[b]).wait()

        def transpose(b, ncb):
            vi = vin.at[b]
            vo = vout.at[b]

            @pl.loop(0, ncb)
            def _(c0):
                cv = lanes + c0 * 16
                sv = sq + c0 * 4
                for f0 in range(0, F, 16):
                    vs = [plsc.load_gather(vi, [fd[d] + f0, cv])
                          for d in range(16)]
                    for d in range(16):
                        plsc.store_scatter(vo, [sv, wq + fd[d] + f0], vs[d])

        start_in(0, 0)

        @pl.loop(0, nw)
        def _(m):
            b = lax.rem(m, NBUF)

            @pl.when(m >= NBUF)
            def _():
                wait_out(b)

            wait_in(b)

            @pl.when(m + 1 < nw)
            def _():
                start_in(m + 1, 1 - b)

            transpose(b, LANE // 16)
            start_out(m, b)

        wait_out(lax.rem(nw - 1, NBUF))
        wait_out(lax.rem(nw, NBUF))

        if TAIL:
            trows = TAIL * F // LANE

            @pl.when(w == NB % NW)
            def _():
                pltpu.sync_copy(tail, vin.at[0, pl.ds(0, trows), :])
                pltpu.sync_copy(vin.at[0, pl.ds(0, trows), :],
                                dst.at[pl.ds(NB * F, trows), :])

    return repack_kernel


@functools.lru_cache(maxsize=None)
def _build(nb, nf, D):
    B = nb * nf
    nsub = D // 8
    tile = 8 * LANE
    assert D % 16 == 0 and nb % LANE == 0 and B % (NW * K * LANE) == 0
    nblk = B // (NW * LANE)
    ng = nblk // K
    tpj = nb // LANE
    assert ng >= 2 and ng % 2 == 0

    mesh = plsc.VectorSubcoreMesh(core_axis_name="c", subcore_axis_name="s")

    @functools.partial(
        pl.kernel,
        out_type=jax.ShapeDtypeStruct((nf * nsub, tpj * tile), jnp.float32),
        mesh=mesh,
        scratch_types=[
            pltpu.VMEM((nblk * LANE,), jnp.int32),
            pltpu.VMEM((NBUF, K * LANE, D), jnp.float32),
            pltpu.VMEM((NBUF, K, nsub, tile), jnp.float32),
            pltpu.SemaphoreType.DMA((NBUF,)),
            pltpu.SemaphoreType.DMA((NBUF,)),
        ],
        compiler_params=pltpu.CompilerParams(use_tc_tiling_on_sc=False,
                                             needs_layout_passes=False),
    )
    def gather_kernel(table, idx_hbm, out_hbm, idx_v, rows, tiles, gsem, ssem):
        w = lax.axis_index("s") * 2 + lax.axis_index("c")
        pltpu.sync_copy(idx_hbm.at[pl.ds(w * nblk * LANE, nblk * LANE)],
                        idx_v)
        gbase = w * ng
        lanes = lax.broadcasted_iota(jnp.int32, (16,), 0)

        def start_gather(g_local, b):
            pltpu.async_copy(
                table.at[idx_v.at[pl.ds(g_local * K * LANE, K * LANE)]],
                rows.at[b], gsem.at[b])

        def wait_gather(b):
            pltpu.make_async_copy(table.at[idx_v.at[pl.ds(0, K * LANE)]],
                                  rows.at[b], gsem.at[b]).wait()

        def transpose(b, u):
            rr = rows.at[b]
            tt = tiles.at[b, u]
            fd = [lax.rem(lanes + d, 16) for d in range(16)]
            wd = [lax.rem(fd[d], 8) * LANE + lanes for d in range(16)]

            @pl.loop(0, LANE // 16)
            def _(k):
                bv = lanes + u * LANE + k * 16
                for f0 in range(0, D, 16):
                    vs = [plsc.load_gather(rr, [bv, fd[d] + f0])
                          for d in range(16)]
                    for d in range(16):
                        plsc.store_scatter(
                            tt, [fd[d] // 8 + f0 // 8, wd[d] + k * 16], vs[d])

        def process(g, b):
            for u in range(K):
                c = g * K + u
                j = c // tpj
                t = lax.rem(c, tpj)
                transpose(b, u)
                pltpu.async_copy(
                    tiles.at[b, u],
                    out_hbm.at[pl.ds(j * nsub, nsub), pl.ds(t * tile, tile)],
                    ssem.at[b])

        def wait_scatter(b):
            for u in range(K):
                pltpu.make_async_copy(
                    tiles.at[b, u],
                    out_hbm.at[pl.ds(0, nsub), pl.ds(0, tile)],
                    ssem.at[b]).wait()

        start_gather(0, 0)

        @pl.loop(0, ng)
        def _(g_local):
            b = lax.rem(g_local, NBUF)

            @pl.when(g_local >= NBUF)
            def _():
                wait_scatter(b)

            wait_gather(b)

            @pl.when(g_local + 1 < ng)
            def _():
                start_gather(g_local + 1, 1 - b)

            process(gbase + g_local, b)

        wait_scatter(lax.rem(ng - 1, NBUF))
        wait_scatter(lax.rem(ng, NBUF))

    return gather_kernel


def kernel(image, gather_indices):
    nb, nf, _ = gather_indices.shape
    B = nb * nf
    D = image.shape[1]
    idx = jnp.transpose(gather_indices, (1, 2, 0)).reshape(B).astype(jnp.int32)
    nrow = image.shape[0]
    tail_n = nrow % LANE
    tail2 = jnp.reshape(image[nrow - tail_n:, :], (tail_n * D // LANE, LANE))
    lin = _build_repack(D, nrow)(jnp.transpose(image), tail2)
    table = jnp.reshape(lin, (nrow, D))
    outb = _build(nb, nf, D)(table, idx)
    out5 = outb.reshape(nf, D // 8, nb // LANE, 8, LANE)
    return jnp.transpose(out5, (2, 4, 0, 1, 3)).reshape(nb, nf, D)
